# baseline (speedup 1.0000x reference)
import jax
import jax.numpy as jnp
from jax.experimental import pallas as pl
from jax.experimental.pallas import tpu as pltpu

_D_MODEL = 768
_N_EXPERTS = 8
_TOP_K = 2
_BLOCK = 4096


def _router_kernel(h_ref, w_ref, b_ref, idx_ref, wt_out_ref, logits_ref):
    logits_t = jax.lax.dot_general(
        w_ref[...], h_ref[...],
        (((1,), (1,)), ((), ())),
        preferred_element_type=jnp.float32,
    ) + b_ref[...]
    logits_ref[...] = logits_t

    i1 = jnp.argmax(logits_t, axis=0).astype(jnp.int32)
    v1 = jnp.max(logits_t, axis=0)
    expert_ids = jax.lax.broadcasted_iota(jnp.int32, logits_t.shape, 0)
    masked = jnp.where(expert_ids == i1[None, :], -jnp.inf, logits_t)
    i2 = jnp.argmax(masked, axis=0).astype(jnp.int32)
    v2 = jnp.max(masked, axis=0)

    idx_ref[...] = jnp.stack([i1, i2], axis=0)

    e2 = jnp.exp(v2 - v1)
    denom = 1.0 + e2
    wt_out_ref[...] = jnp.stack([1.0 / denom, e2 / denom], axis=0)


@jax.jit
def kernel(h, W, b):
    n_tokens = h.shape[0]
    grid = (n_tokens // _BLOCK,)
    b2 = b.reshape(_N_EXPERTS, 1)
    idx_t, w_t, logits_t = pl.pallas_call(
        _router_kernel,
        grid=grid,
        in_specs=[
            pl.BlockSpec((_BLOCK, _D_MODEL), lambda i: (i, 0)),
            pl.BlockSpec((_N_EXPERTS, _D_MODEL), lambda i: (0, 0)),
            pl.BlockSpec((_N_EXPERTS, 1), lambda i: (0, 0)),
        ],
        out_specs=[
            pl.BlockSpec((_TOP_K, _BLOCK), lambda i: (0, i)),
            pl.BlockSpec((_TOP_K, _BLOCK), lambda i: (0, i)),
            pl.BlockSpec((_N_EXPERTS, _BLOCK), lambda i: (0, i)),
        ],
        out_shape=[
            jax.ShapeDtypeStruct((_TOP_K, n_tokens), jnp.int32),
            jax.ShapeDtypeStruct((_TOP_K, n_tokens), jnp.float32),
            jax.ShapeDtypeStruct((_N_EXPERTS, n_tokens), jnp.float32),
        ],
        compiler_params=pltpu.CompilerParams(
            dimension_semantics=("arbitrary",),
        ),
    )(h, W, b2)
    return (idx_t.T, w_t.T, logits_t.T)
